# baseline (speedup 1.0000x reference)
import jax
import jax.numpy as jnp
from jax.experimental import pallas as pl

_BS, _NV, _NP, _FD = 64, 4, 512, 10
_THR = 6.0
_SCALE = 1.0


def _chamfer_body(goal_ref, state_ref, out_ref):
    acc = None
    for v in range(_NV):
        g = goal_ref[0, v]
        s = state_ref[0, v]

        def srow(f):
            return jnp.reshape(s[:, f:f + 1], (1, _NP))

        P = None
        for f in range(5, 9):
            d = g[:, f:f + 1] - srow(f)
            P = d * d if P is None else P + d * d

        ex = g[:, 0:1] - srow(0)
        ey = g[:, 1:2] - srow(1)
        D2 = ex * ex + ey * ey

        minv_g = jnp.min(P, axis=1, keepdims=True)
        sel = P == minv_g
        q1 = jnp.sum(jnp.where(sel, D2, 0.0), axis=1, keepdims=True)
        q1r = jnp.reshape(q1, (1, _NP))
        m1r = jnp.reshape(minv_g, (1, _NP))
        xy1 = jnp.where(m1r > _THR, 1.0, jnp.sqrt(q1r))

        minv_s = jnp.min(P, axis=0, keepdims=True)
        sel2 = P == minv_s
        q2 = jnp.sum(jnp.where(sel2, D2, 0.0), axis=0, keepdims=True)
        xy2 = jnp.where(minv_s > _THR, 1.0, jnp.sqrt(q2))

        part = xy1 + xy2
        acc = part if acc is None else acc + part

    total = jnp.sum(acc)
    out_ref[...] = (total * (-_SCALE / (2.0 * _NP * _NV))).reshape(1, 1, 1)


@jax.jit
def kernel(achieved_goal, desired_goal):
    out = pl.pallas_call(
        _chamfer_body,
        grid=(_BS,),
        in_specs=[
            pl.BlockSpec((1, _NV, _NP, _FD), lambda b: (b, 0, 0, 0)),
            pl.BlockSpec((1, _NV, _NP, _FD), lambda b: (b, 0, 0, 0)),
        ],
        out_specs=pl.BlockSpec((1, 1, 1), lambda b: (b, 0, 0)),
        out_shape=jax.ShapeDtypeStruct((_BS, 1, 1), jnp.float32),
    )(desired_goal, achieved_goal)
    return out.reshape(_BS, 1)


# speedup vs baseline: 1.7033x; 1.7033x over previous
import jax
import jax.numpy as jnp
from jax.experimental import pallas as pl

_BS, _NV, _NP, _FD = 64, 4, 512, 10
_THR = 6.0
_SCALE = 1.0


def _chamfer_body(goal_ref, stateT_ref, out_ref):
    acc = None
    for v in range(_NV):
        g = goal_ref[0, v]

        def srow(f):
            return stateT_ref[f:f + 1, 0, v, :]

        P = None
        for f in range(5, 9):
            d = g[:, f:f + 1] - srow(f)
            P = d * d if P is None else P + d * d

        ex = g[:, 0:1] - srow(0)
        ey = g[:, 1:2] - srow(1)
        D2 = ex * ex + ey * ey

        minv_g = jnp.min(P, axis=1, keepdims=True)
        sel = P == minv_g
        q1 = jnp.sum(jnp.where(sel, D2, 0.0), axis=1, keepdims=True)
        q1r = jnp.reshape(q1, (1, _NP))
        m1r = jnp.reshape(minv_g, (1, _NP))
        xy1 = jnp.where(m1r > _THR, 1.0, jnp.sqrt(q1r))

        minv_s = jnp.min(P, axis=0, keepdims=True)
        sel2 = P == minv_s
        q2 = jnp.sum(jnp.where(sel2, D2, 0.0), axis=0, keepdims=True)
        xy2 = jnp.where(minv_s > _THR, 1.0, jnp.sqrt(q2))

        part = xy1 + xy2
        acc = part if acc is None else acc + part

    total = jnp.sum(acc)
    out_ref[...] = (total * (-_SCALE / (2.0 * _NP * _NV))).reshape(1, 1, 1)


@jax.jit
def kernel(achieved_goal, desired_goal):
    stateT = jnp.moveaxis(achieved_goal, -1, 0)
    out = pl.pallas_call(
        _chamfer_body,
        grid=(_BS,),
        in_specs=[
            pl.BlockSpec((1, _NV, _NP, _FD), lambda b: (b, 0, 0, 0)),
            pl.BlockSpec((_FD, 1, _NV, _NP), lambda b: (0, b, 0, 0)),
        ],
        out_specs=pl.BlockSpec((1, 1, 1), lambda b: (b, 0, 0)),
        out_shape=jax.ShapeDtypeStruct((_BS, 1, 1), jnp.float32),
    )(desired_goal, stateT)
    return out.reshape(_BS, 1)
